# baseline (speedup 1.0000x reference)
import functools
import math

import jax
import jax.numpy as jnp
from jax.experimental import pallas as pl
from jax.experimental.pallas import tpu as pltpu


def _dot(a, b):
    return jax.lax.dot_general(a, b, (((1,), (0,)), ((), ())),
                               preferred_element_type=jnp.float32)


def _dot_t(a, b):
    return jax.lax.dot_general(a, b, (((1,), (1,)), ((), ())),
                               preferred_element_type=jnp.float32)


def _moe_body(cap, x_ref, wg_ref, we_ref, be_ref, u_ref, out_ref, cnt_ref):
    i = pl.program_id(0)

    @pl.when(i == 0)
    def _():
        cnt_ref[...] = jnp.zeros_like(cnt_ref)

    xb_bf = x_ref[...].astype(jnp.bfloat16)
    E = wg_ref.shape[0]
    B = xb_bf.shape[0]

    logits = _dot_t(wg_ref[...], xb_bf)
    m = jnp.max(logits, axis=0, keepdims=True)
    g = jnp.exp(logits - m)
    gates = g / jnp.sum(g, axis=0, keepdims=True)
    gate = jnp.max(gates, axis=0, keepdims=True)

    s_iota = jax.lax.broadcasted_iota(jnp.int32, (E, B), 0)
    first = jnp.min(jnp.where(gates == gate, s_iota, E), axis=0, keepdims=True)
    mask = (s_iota == first).astype(jnp.float32)

    mask_bf = mask.astype(jnp.bfloat16)
    u = u_ref[...]
    off = cnt_ref[...]
    pos_chunks = []
    for k in range(B // 128):
        pc = _dot(mask_bf[:, k * 128:(k + 1) * 128], u)
        pos_chunks.append(pc + (off - 1.0))
        off = off + pc[:, 127:128]
    cnt_ref[...] = off
    pos = jnp.concatenate(pos_chunks, axis=1)

    keep = jnp.sum(jnp.where(pos < cap, mask, 0.0), axis=0, keepdims=True)
    coef = mask * (gate * keep)

    acc = jnp.zeros((we_ref.shape[1], B), jnp.float32)
    for e in range(E):
        ye = _dot_t(we_ref[e], xb_bf)
        acc = acc + coef[e:e + 1, :] * (ye + be_ref[:, e:e + 1])
    out_ref[...] = acc.T


def kernel(inputs, Wg, We, be):
    d = inputs.shape[-1]
    E = Wg.shape[1]
    x = inputs.reshape(-1, d)
    T = x.shape[0]
    cap = float(math.ceil(T / E))
    B = 1024
    nblocks = T // B

    WgT = Wg.T.astype(jnp.bfloat16)
    WeT = We.transpose(0, 2, 1).astype(jnp.bfloat16)
    beT = be.T
    u = jnp.triu(jnp.ones((128, 128), jnp.bfloat16))

    out = pl.pallas_call(
        functools.partial(_moe_body, cap),
        grid=(nblocks,),
        in_specs=[
            pl.BlockSpec((B, d), lambda i: (i, 0)),
            pl.BlockSpec((E, d), lambda i: (0, 0)),
            pl.BlockSpec((E, d, d), lambda i: (0, 0, 0)),
            pl.BlockSpec((d, E), lambda i: (0, 0)),
            pl.BlockSpec((128, 128), lambda i: (0, 0)),
        ],
        out_specs=pl.BlockSpec((B, d), lambda i: (i, 0)),
        out_shape=jax.ShapeDtypeStruct((T, d), jnp.float32),
        scratch_shapes=[pltpu.VMEM((E, 1), jnp.float32)],
        compiler_params=pltpu.CompilerParams(
            dimension_semantics=("arbitrary",)),
    )(x, WgT, WeT, beT, u)
    return out.reshape(inputs.shape)


# speedup vs baseline: 2.5047x; 2.5047x over previous
import functools
import math

import jax
import jax.numpy as jnp
from jax.experimental import pallas as pl
from jax.experimental.pallas import tpu as pltpu


def _dot(a, b):
    return jax.lax.dot_general(a, b, (((1,), (0,)), ((), ())),
                               preferred_element_type=jnp.float32)


def _moe_body(cap, x_ref, wg_ref, we_ref, be_ref, u_ref, out_ref, cnt_ref):
    i = pl.program_id(0)

    @pl.when(i == 0)
    def _():
        cnt_ref[...] = jnp.zeros_like(cnt_ref)

    xb = x_ref[...]
    xb_bf = xb.astype(jnp.bfloat16)
    E = wg_ref.shape[0]
    B = xb.shape[1]

    logits = _dot(wg_ref[...], xb_bf)
    m = jnp.max(logits, axis=0, keepdims=True)
    g = jnp.exp(logits - m)
    gates = g / jnp.sum(g, axis=0, keepdims=True)
    gate = jnp.max(gates, axis=0, keepdims=True)

    s_iota = jax.lax.broadcasted_iota(jnp.int32, (E, B), 0)
    first = jnp.min(jnp.where(gates == gate, s_iota, E), axis=0, keepdims=True)
    mask = (s_iota == first).astype(jnp.float32)

    mask_bf = mask.astype(jnp.bfloat16)
    u = u_ref[...]
    off = cnt_ref[...]
    pos_chunks = []
    for k in range(B // 128):
        pc = _dot(mask_bf[:, k * 128:(k + 1) * 128], u)
        pos_chunks.append(pc + (off - 1.0))
        off = off + pc[:, 127:128]
    cnt_ref[...] = off
    pos = jnp.concatenate(pos_chunks, axis=1)

    keep = jnp.sum(jnp.where(pos < cap, mask, 0.0), axis=0, keepdims=True)
    coef = mask * (gate * keep)

    acc = jnp.zeros(xb.shape, jnp.float32)
    for e in range(E):
        ye = _dot(we_ref[e], xb_bf)
        acc = acc + coef[e:e + 1, :] * (ye + be_ref[:, e:e + 1])
    out_ref[...] = acc


def kernel(inputs, Wg, We, be):
    d = inputs.shape[-1]
    E = Wg.shape[1]
    x = inputs.reshape(-1, d)
    T = x.shape[0]
    cap = float(math.ceil(T / E))
    B = 4096
    nblocks = T // B

    x_T = x.T
    WgT = Wg.T.astype(jnp.bfloat16)
    WeT = We.transpose(0, 2, 1).astype(jnp.bfloat16)
    beT = be.T
    u = jnp.triu(jnp.ones((128, 128), jnp.bfloat16))

    out_T = pl.pallas_call(
        functools.partial(_moe_body, cap),
        grid=(nblocks,),
        in_specs=[
            pl.BlockSpec((d, B), lambda i: (0, i)),
            pl.BlockSpec((E, d), lambda i: (0, 0)),
            pl.BlockSpec((E, d, d), lambda i: (0, 0, 0)),
            pl.BlockSpec((d, E), lambda i: (0, 0)),
            pl.BlockSpec((128, 128), lambda i: (0, 0)),
        ],
        out_specs=pl.BlockSpec((d, B), lambda i: (0, i)),
        out_shape=jax.ShapeDtypeStruct((d, T), jnp.float32),
        scratch_shapes=[pltpu.VMEM((E, 1), jnp.float32)],
        compiler_params=pltpu.CompilerParams(
            dimension_semantics=("arbitrary",)),
    )(x_T, WgT, WeT, beT, u)
    return out_T.T.reshape(inputs.shape)


# speedup vs baseline: 2.6396x; 1.0539x over previous
import functools
import math

import jax
import jax.numpy as jnp
from jax.experimental import pallas as pl
from jax.experimental.pallas import tpu as pltpu


def _dot(a, b):
    return jax.lax.dot_general(a, b, (((1,), (0,)), ((), ())),
                               preferred_element_type=jnp.float32)


def _moe_body(cap, x_ref, wg_ref, we_ref, be_ref, u_ref, out_ref, cnt_ref):
    i = pl.program_id(0)

    @pl.when(i == 0)
    def _():
        cnt_ref[...] = jnp.zeros_like(cnt_ref)

    xb = x_ref[...]
    xb_bf = xb.astype(jnp.bfloat16)
    E = wg_ref.shape[0]
    B = xb.shape[1]

    logits = _dot(wg_ref[...], xb_bf)
    m = jnp.max(logits, axis=0, keepdims=True)
    g = jnp.exp(logits - m)
    gates = g / jnp.sum(g, axis=0, keepdims=True)
    gate = jnp.max(gates, axis=0, keepdims=True)

    s_iota = jax.lax.broadcasted_iota(jnp.int32, (E, B), 0)
    first = jnp.min(jnp.where(gates == gate, s_iota, E), axis=0, keepdims=True)
    mask = (s_iota == first).astype(jnp.float32)

    mask_bf = mask.astype(jnp.bfloat16)
    u = u_ref[...]
    off = cnt_ref[...]
    pos_chunks = []
    for k in range(B // 128):
        pc = _dot(mask_bf[:, k * 128:(k + 1) * 128], u)
        pos_chunks.append(pc + (off - 1.0))
        off = off + pc[:, 127:128]
    cnt_ref[...] = off
    pos = jnp.concatenate(pos_chunks, axis=1)

    keep = jnp.sum(jnp.where(pos < cap, mask, 0.0), axis=0, keepdims=True)
    coef = mask * (gate * keep)

    acc = jnp.zeros(xb.shape, jnp.float32)
    for e in range(E):
        ye = _dot(we_ref[e], xb_bf)
        acc = acc + coef[e:e + 1, :] * (ye + be_ref[:, e:e + 1])
    out_ref[...] = acc


def kernel(inputs, Wg, We, be):
    d = inputs.shape[-1]
    E = Wg.shape[1]
    x = inputs.reshape(-1, d)
    T = x.shape[0]
    cap = float(math.ceil(T / E))
    B = 8192
    nblocks = T // B

    x_T = x.T
    WgT = Wg.T.astype(jnp.bfloat16)
    WeT = We.transpose(0, 2, 1).astype(jnp.bfloat16)
    beT = be.T
    u = jnp.triu(jnp.ones((128, 128), jnp.bfloat16))

    out_T = pl.pallas_call(
        functools.partial(_moe_body, cap),
        grid=(nblocks,),
        in_specs=[
            pl.BlockSpec((d, B), lambda i: (0, i)),
            pl.BlockSpec((E, d), lambda i: (0, 0)),
            pl.BlockSpec((E, d, d), lambda i: (0, 0, 0)),
            pl.BlockSpec((d, E), lambda i: (0, 0)),
            pl.BlockSpec((128, 128), lambda i: (0, 0)),
        ],
        out_specs=pl.BlockSpec((d, B), lambda i: (0, i)),
        out_shape=jax.ShapeDtypeStruct((d, T), jnp.float32),
        scratch_shapes=[pltpu.VMEM((E, 1), jnp.float32)],
        compiler_params=pltpu.CompilerParams(
            dimension_semantics=("arbitrary",)),
    )(x_T, WgT, WeT, beT, u)
    return out_T.T.reshape(inputs.shape)
